# baseline (speedup 1.0000x reference)
import functools

import jax
import jax.numpy as jnp
import numpy as np
from jax import lax
from jax.experimental import pallas as pl
from jax.experimental.pallas import tpu as pltpu
from jax.experimental.pallas import tpu_sc as plsc

NC = 2
NS = 16
NW = NC * NS
BATCH = 128
NROW = 3
NIDX = 6


def _sc_partials(n_nodes, d_feat, n_edges):
    assert n_edges % BATCH == 0
    nbatch = n_edges // BATCH
    nb_lo = nbatch // NW
    n_extra = nbatch - nb_lo * NW
    assert nb_lo % NIDX == 0
    ngroups = nb_lo // NIDX
    rpt = (n_nodes // NS) // 8 * 8
    rtail = n_nodes - rpt * NS

    mesh = plsc.VectorSubcoreMesh(core_axis_name="c", subcore_axis_name="s")

    scratch = (
        [pltpu.VMEM_SHARED((n_nodes, d_feat), jnp.float32)]
        + [pltpu.VMEM((2, BATCH), jnp.int32) for _ in range(NIDX)]
        + [pltpu.VMEM((BATCH, d_feat), jnp.float32) for _ in range(NROW)]
        + [pltpu.SemaphoreType.DMA for _ in range(NIDX + 2 * NROW)]
    )

    @functools.partial(
        pl.kernel,
        out_type=jax.ShapeDtypeStruct((NC, n_nodes, d_feat), jnp.float32),
        mesh=mesh,
        scratch_types=scratch,
    )
    def run(x_hbm, pk_hbm, zero_hbm, part_hbm, acc, *bufs):
        idx = bufs[:NIDX]
        rows = bufs[NIDX:NIDX + NROW]
        semi = bufs[NIDX + NROW:2 * NIDX + NROW]
        semg = bufs[2 * NIDX + NROW:2 * NIDX + 2 * NROW]
        sems = bufs[2 * NIDX + 2 * NROW:2 * NIDX + 3 * NROW]
        c = lax.axis_index("c")
        s = lax.axis_index("s")
        w = s * NC + c
        start = w * nb_lo + jnp.minimum(w, n_extra)

        for p in range(2):
            pltpu.async_copy(pk_hbm.at[:, start + p], idx[p], semi[p])
        pltpu.sync_copy(zero_hbm.at[pl.ds(s * rpt, rpt)],
                        acc.at[pl.ds(s * rpt, rpt)])
        if rtail:
            @pl.when(s == 0)
            def _():
                pltpu.sync_copy(zero_hbm.at[pl.ds(rpt * NS, rtail)],
                                acc.at[pl.ds(rpt * NS, rtail)])
        plsc.subcore_barrier()

        pltpu.make_async_copy(pk_hbm.at[:, start], idx[0], semi[0]).wait()
        pltpu.async_copy(x_hbm.at[idx[0].at[0]], rows[0], semg[0])

        def group(g, _):
            base = start + g * NIDX
            for p in range(NIDX):
                pr = p % NROW
                rn = (p + 1) % NROW
                pn = (p + 1) % NIDX
                def issue_next():
                    pltpu.make_async_copy(pk_hbm.at[:, base + p + 1], idx[pn],
                                          semi[pn]).wait()
                    if p >= 2:
                        pltpu.make_async_copy(x_hbm.at[pl.ds(0, BATCH)],
                                              rows[rn], sems[rn]).wait()
                    else:
                        @pl.when(g > 0)
                        def _():
                            pltpu.make_async_copy(
                                x_hbm.at[pl.ds(0, BATCH)],
                                rows[rn], sems[rn]).wait()
                    pltpu.async_copy(x_hbm.at[idx[pn].at[0]], rows[rn],
                                     semg[rn])
                    pf = (p + 2) % NIDX
                    if p <= NIDX - 3:
                        pltpu.async_copy(pk_hbm.at[:, base + p + 2], idx[pf],
                                         semi[pf])
                    else:
                        @pl.when(g < ngroups - 1)
                        def _():
                            pltpu.async_copy(pk_hbm.at[:, base + p + 2],
                                             idx[pf], semi[pf])

                if p < NIDX - 1:
                    issue_next()
                else:
                    @pl.when(g < ngroups - 1)
                    def _():
                        issue_next()
                pltpu.make_async_copy(x_hbm.at[pl.ds(0, BATCH)], rows[pr],
                                      semg[pr]).wait()
                pltpu.async_copy(rows[pr], acc.at[idx[p].at[1]], sems[pr],
                                 add=True)
            return _

        lax.fori_loop(0, ngroups, group, None)
        for pr in range(NROW):
            pltpu.make_async_copy(x_hbm.at[pl.ds(0, BATCH)],
                                  rows[pr], sems[pr]).wait()

        if n_extra:
            @pl.when(w < n_extra)
            def _():
                bb = start + nb_lo
                pltpu.async_copy(pk_hbm.at[:, bb], idx[0], semi[0]).wait()
                pltpu.async_copy(x_hbm.at[idx[0].at[0]], rows[0],
                                 semg[0]).wait()
                pltpu.async_copy(rows[0], acc.at[idx[0].at[1]], sems[0],
                                 add=True).wait()

        plsc.subcore_barrier()
        pltpu.sync_copy(acc.at[pl.ds(s * rpt, rpt)],
                        part_hbm.at[c, pl.ds(s * rpt, rpt)])
        if rtail:
            @pl.when(s == 0)
            def _():
                pltpu.sync_copy(acc.at[pl.ds(rpt * NS, rtail)],
                                part_hbm.at[c, pl.ds(rpt * NS, rtail)])

    return run


def _tc_add(a, b):
    n_nodes, d_feat = a.shape
    blk = 1000
    grid = n_nodes // blk

    def body(a_ref, b_ref, o_ref):
        o_ref[...] = a_ref[...] + b_ref[...]

    return pl.pallas_call(
        body,
        grid=(grid,),
        in_specs=[pl.BlockSpec((blk, d_feat), lambda i: (i, 0))] * 2,
        out_specs=pl.BlockSpec((blk, d_feat), lambda i: (i, 0)),
        out_shape=jax.ShapeDtypeStruct((n_nodes, d_feat), jnp.float32),
    )(a, b)


def kernel(boundary_x, boundary_index, out_size):
    n_nodes, d_feat = boundary_x.shape
    n_edges = boundary_index.shape[1]
    nbatch = n_edges // BATCH
    packed = boundary_index.astype(jnp.int32).reshape(2, nbatch, BATCH)
    zeros = jnp.asarray(np.zeros((n_nodes, d_feat), np.float32))
    part = _sc_partials(n_nodes, d_feat, n_edges)(boundary_x, packed, zeros)
    return _tc_add(part[0], part[1])


# speedup vs baseline: 1.0352x; 1.0352x over previous
import functools

import jax
import jax.numpy as jnp
import numpy as np
from jax import lax
from jax.experimental import pallas as pl
from jax.experimental.pallas import tpu as pltpu
from jax.experimental.pallas import tpu_sc as plsc

NC = 2
NS = 16
NW = NC * NS
BATCH = 128
NROW = 3
NIDX = 6


def _sc_partials(n_nodes, d_feat, n_edges):
    assert n_edges % BATCH == 0
    nbatch = n_edges // BATCH
    nb_lo = nbatch // NW
    n_extra = nbatch - nb_lo * NW
    assert nb_lo % NIDX == 0
    ngroups = nb_lo // NIDX
    rpt = (n_nodes // NS) // 8 * 8
    rtail = n_nodes - rpt * NS

    mesh = plsc.VectorSubcoreMesh(core_axis_name="c", subcore_axis_name="s")

    scratch = (
        [pltpu.VMEM_SHARED((n_nodes, d_feat), jnp.float32)]
        + [pltpu.VMEM((2, BATCH), jnp.int32) for _ in range(NIDX)]
        + [pltpu.VMEM((BATCH, d_feat), jnp.float32) for _ in range(NROW)]
        + [pltpu.SemaphoreType.DMA for _ in range(NIDX + 2 * NROW)]
    )

    @functools.partial(
        pl.kernel,
        out_type=jax.ShapeDtypeStruct((NC, n_nodes, d_feat), jnp.float32),
        mesh=mesh,
        scratch_types=scratch,
    )
    def run(x_hbm, pk_hbm, zero_hbm, part_hbm, acc, *bufs):
        idx = bufs[:NIDX]
        rows = bufs[NIDX:NIDX + NROW]
        semi = bufs[NIDX + NROW:2 * NIDX + NROW]
        semg = bufs[2 * NIDX + NROW:2 * NIDX + 2 * NROW]
        sems = bufs[2 * NIDX + 2 * NROW:2 * NIDX + 3 * NROW]
        c = lax.axis_index("c")
        s = lax.axis_index("s")
        w = s * NC + c
        start = w * nb_lo + jnp.minimum(w, n_extra)

        for p in range(2):
            pltpu.async_copy(pk_hbm.at[start + p], idx[p], semi[p])
        pltpu.sync_copy(zero_hbm.at[pl.ds(s * rpt, rpt)],
                        acc.at[pl.ds(s * rpt, rpt)])
        if rtail:
            @pl.when(s == 0)
            def _():
                pltpu.sync_copy(zero_hbm.at[pl.ds(rpt * NS, rtail)],
                                acc.at[pl.ds(rpt * NS, rtail)])
        plsc.subcore_barrier()

        pltpu.make_async_copy(pk_hbm.at[start], idx[0], semi[0]).wait()
        pltpu.async_copy(x_hbm.at[idx[0].at[0]], rows[0], semg[0])

        def group(g, _):
            base = start + g * NIDX
            for p in range(NIDX):
                pr = p % NROW
                rn = (p + 1) % NROW
                pn = (p + 1) % NIDX
                def issue_next():
                    pltpu.make_async_copy(pk_hbm.at[base + p + 1], idx[pn],
                                          semi[pn]).wait()
                    if p >= 2:
                        pltpu.make_async_copy(x_hbm.at[pl.ds(0, BATCH)],
                                              rows[rn], sems[rn]).wait()
                    else:
                        @pl.when(g > 0)
                        def _():
                            pltpu.make_async_copy(
                                x_hbm.at[pl.ds(0, BATCH)],
                                rows[rn], sems[rn]).wait()
                    pltpu.async_copy(x_hbm.at[idx[pn].at[0]], rows[rn],
                                     semg[rn])
                    pf = (p + 2) % NIDX
                    if p <= NIDX - 3:
                        pltpu.async_copy(pk_hbm.at[base + p + 2], idx[pf],
                                         semi[pf])
                    else:
                        @pl.when(g < ngroups - 1)
                        def _():
                            pltpu.async_copy(pk_hbm.at[base + p + 2],
                                             idx[pf], semi[pf])

                if p < NIDX - 1:
                    issue_next()
                else:
                    @pl.when(g < ngroups - 1)
                    def _():
                        issue_next()
                pltpu.make_async_copy(x_hbm.at[pl.ds(0, BATCH)], rows[pr],
                                      semg[pr]).wait()
                pltpu.async_copy(rows[pr], acc.at[idx[p].at[1]], sems[pr],
                                 add=True)
            return _

        lax.fori_loop(0, ngroups, group, None)
        for pr in range(NROW):
            pltpu.make_async_copy(x_hbm.at[pl.ds(0, BATCH)],
                                  rows[pr], sems[pr]).wait()

        if n_extra:
            @pl.when(w < n_extra)
            def _():
                bb = start + nb_lo
                pltpu.async_copy(pk_hbm.at[bb], idx[0], semi[0]).wait()
                pltpu.async_copy(x_hbm.at[idx[0].at[0]], rows[0],
                                 semg[0]).wait()
                pltpu.async_copy(rows[0], acc.at[idx[0].at[1]], sems[0],
                                 add=True).wait()

        plsc.subcore_barrier()
        pltpu.sync_copy(acc.at[pl.ds(s * rpt, rpt)],
                        part_hbm.at[c, pl.ds(s * rpt, rpt)])
        if rtail:
            @pl.when(s == 0)
            def _():
                pltpu.sync_copy(acc.at[pl.ds(rpt * NS, rtail)],
                                part_hbm.at[c, pl.ds(rpt * NS, rtail)])

    return run


def _tc_add(a, b):
    n_nodes, d_feat = a.shape
    blk = 1000
    grid = n_nodes // blk

    def body(a_ref, b_ref, o_ref):
        o_ref[...] = a_ref[...] + b_ref[...]

    return pl.pallas_call(
        body,
        grid=(grid,),
        in_specs=[pl.BlockSpec((blk, d_feat), lambda i: (i, 0))] * 2,
        out_specs=pl.BlockSpec((blk, d_feat), lambda i: (i, 0)),
        out_shape=jax.ShapeDtypeStruct((n_nodes, d_feat), jnp.float32),
    )(a, b)


def kernel(boundary_x, boundary_index, out_size):
    n_nodes, d_feat = boundary_x.shape
    n_edges = boundary_index.shape[1]
    nbatch = n_edges // BATCH
    packed = boundary_index.astype(jnp.int32).reshape(2, nbatch, BATCH)
    packed = packed.transpose(1, 0, 2)
    zeros = jnp.asarray(np.zeros((n_nodes, d_feat), np.float32))
    part = _sc_partials(n_nodes, d_feat, n_edges)(boundary_x, packed, zeros)
    return _tc_add(part[0], part[1])


# speedup vs baseline: 1.0499x; 1.0141x over previous
import functools

import jax
import jax.numpy as jnp
import numpy as np
from jax import lax
from jax.experimental import pallas as pl
from jax.experimental.pallas import tpu as pltpu
from jax.experimental.pallas import tpu_sc as plsc

NC = 2
NS = 16
NW = NC * NS
BATCH = 128
NROW = 3
NIDX = 6


def _sc_partials(n_nodes, d_feat, n_edges):
    assert n_edges % BATCH == 0
    nbatch = n_edges // BATCH
    nb_lo = nbatch // NW
    n_extra = nbatch - nb_lo * NW
    assert nb_lo % NIDX == 0
    ngroups = nb_lo // NIDX
    rpt = (n_nodes // NS) // 8 * 8
    rtail = n_nodes - rpt * NS

    mesh = plsc.VectorSubcoreMesh(core_axis_name="c", subcore_axis_name="s")

    scratch = (
        [pltpu.VMEM_SHARED((n_nodes, d_feat), jnp.float32)]
        + [pltpu.VMEM((2, BATCH), jnp.int32) for _ in range(NIDX)]
        + [pltpu.VMEM((BATCH, d_feat), jnp.float32) for _ in range(NROW)]
        + [pltpu.SemaphoreType.DMA for _ in range(NIDX + 2 * NROW)]
    )

    @functools.partial(
        pl.kernel,
        out_type=jax.ShapeDtypeStruct((NC, n_nodes, d_feat), jnp.float32),
        mesh=mesh,
        scratch_types=scratch,
    )
    def run(x_hbm, pk_hbm, zero_hbm, part_hbm, acc, *bufs):
        idx = bufs[:NIDX]
        rows = bufs[NIDX:NIDX + NROW]
        semi = bufs[NIDX + NROW:2 * NIDX + NROW]
        semg = bufs[2 * NIDX + NROW:2 * NIDX + 2 * NROW]
        sems = bufs[2 * NIDX + 2 * NROW:2 * NIDX + 3 * NROW]
        c = lax.axis_index("c")
        s = lax.axis_index("s")
        w = s * NC + c
        start = w * nb_lo + jnp.minimum(w, n_extra)

        for p in range(2):
            pltpu.async_copy(pk_hbm.at[start + p], idx[p], semi[p])
        pltpu.sync_copy(zero_hbm.at[pl.ds(s * rpt, rpt)],
                        acc.at[pl.ds(s * rpt, rpt)])
        if rtail:
            @pl.when(s == 0)
            def _():
                pltpu.sync_copy(zero_hbm.at[pl.ds(rpt * NS, rtail)],
                                acc.at[pl.ds(rpt * NS, rtail)])
        plsc.subcore_barrier()

        pltpu.make_async_copy(pk_hbm.at[start], idx[0], semi[0]).wait()
        pltpu.async_copy(x_hbm.at[idx[0].at[0]], rows[0], semg[0])

        def group(g, _):
            base = start + g * NIDX
            for p in range(NIDX):
                pr = p % NROW
                rn = (p + 1) % NROW
                pn = (p + 1) % NIDX
                def issue_next():
                    pltpu.make_async_copy(pk_hbm.at[base + p + 1], idx[pn],
                                          semi[pn]).wait()
                    if p >= 2:
                        pltpu.make_async_copy(x_hbm.at[pl.ds(0, BATCH)],
                                              rows[rn], sems[rn]).wait()
                    else:
                        @pl.when(g > 0)
                        def _():
                            pltpu.make_async_copy(
                                x_hbm.at[pl.ds(0, BATCH)],
                                rows[rn], sems[rn]).wait()
                    pltpu.async_copy(x_hbm.at[idx[pn].at[0]], rows[rn],
                                     semg[rn])
                    pf = (p + 2) % NIDX
                    if p <= NIDX - 3:
                        pltpu.async_copy(pk_hbm.at[base + p + 2], idx[pf],
                                         semi[pf])
                    else:
                        @pl.when(g < ngroups - 1)
                        def _():
                            pltpu.async_copy(pk_hbm.at[base + p + 2],
                                             idx[pf], semi[pf])

                if p < NIDX - 1:
                    issue_next()
                else:
                    @pl.when(g < ngroups - 1)
                    def _():
                        issue_next()
                pltpu.make_async_copy(x_hbm.at[pl.ds(0, BATCH)], rows[pr],
                                      semg[pr]).wait()
                pltpu.async_copy(rows[pr], acc.at[idx[p].at[1]], sems[pr],
                                 add=True)
            return _

        lax.fori_loop(0, ngroups, group, None)
        for pr in range(NROW):
            pltpu.make_async_copy(x_hbm.at[pl.ds(0, BATCH)],
                                  rows[pr], sems[pr]).wait()

        if n_extra:
            @pl.when(w < n_extra)
            def _():
                bb = start + nb_lo
                pltpu.async_copy(pk_hbm.at[bb], idx[0], semi[0]).wait()
                pltpu.async_copy(x_hbm.at[idx[0].at[0]], rows[0],
                                 semg[0]).wait()
                pltpu.async_copy(rows[0], acc.at[idx[0].at[1]], sems[0],
                                 add=True).wait()

        plsc.subcore_barrier()
        pltpu.sync_copy(acc.at[pl.ds(s * rpt, rpt)],
                        part_hbm.at[c, pl.ds(s * rpt, rpt)])
        if rtail:
            @pl.when(s == 0)
            def _():
                pltpu.sync_copy(acc.at[pl.ds(rpt * NS, rtail)],
                                part_hbm.at[c, pl.ds(rpt * NS, rtail)])

    return run


def _tc_add(a, b):
    n_nodes, d_feat = a.shape
    blk = 2000
    grid = n_nodes // blk

    def body(a_ref, b_ref, o_ref):
        o_ref[...] = a_ref[...] + b_ref[...]

    return pl.pallas_call(
        body,
        grid=(grid,),
        in_specs=[pl.BlockSpec((blk, d_feat), lambda i: (i, 0))] * 2,
        out_specs=pl.BlockSpec((blk, d_feat), lambda i: (i, 0)),
        out_shape=jax.ShapeDtypeStruct((n_nodes, d_feat), jnp.float32),
    )(a, b)


def kernel(boundary_x, boundary_index, out_size):
    n_nodes, d_feat = boundary_x.shape
    n_edges = boundary_index.shape[1]
    nbatch = n_edges // BATCH
    packed = boundary_index.astype(jnp.int32).reshape(2, nbatch, BATCH)
    packed = packed.transpose(1, 0, 2)
    zeros = jnp.asarray(np.zeros((n_nodes, d_feat), np.float32))
    part = _sc_partials(n_nodes, d_feat, n_edges)(boundary_x, packed, zeros)
    return _tc_add(part[0], part[1])
